# baseline (speedup 1.0000x reference)
import functools

import jax
import jax.numpy as jnp
from jax import lax
from jax.experimental import pallas as pl
from jax.experimental.pallas import tpu as pltpu
from jax.experimental.pallas import tpu_sc as plsc

NC = 2
NS = 16
NW = NC * NS


def _sc_mesh():
    return plsc.VectorSubcoreMesh(
        core_axis_name="c", subcore_axis_name="s", num_cores=NC, num_subcores=NS
    )


def _mac(src, dst, sem):
    return pltpu.make_async_copy(src, dst, sem)


def _sc_ep_a(ewb, p, src, dst, zeros, chunk=40):
    e, dm = ewb.shape
    n = zeros.shape[0]
    b_per_w = e // NW
    nch = b_per_w // chunk
    npair = nch // 2
    assert nch % 2 == 0

    buf = lambda: pltpu.VMEM((chunk, dm), jnp.float32)
    ibuf = lambda: pltpu.VMEM((chunk,), jnp.int32)
    sem = pltpu.SemaphoreType.DMA

    @functools.partial(
        pl.kernel,
        mesh=_sc_mesh(),
        out_type=(jax.ShapeDtypeStruct((e, dm), jnp.float32),
                  jax.ShapeDtypeStruct((NC, n, dm), jnp.float32)),
        scratch_types=(
            [buf() for _ in range(4)]
            + [buf() for _ in range(2)]
            + [ibuf() for _ in range(4)]
            + [pltpu.VMEM_SHARED((n, dm), jnp.float32)]
            + [sem] * 12
        ),
    )
    def k(ewb_hbm, p_hbm, src_hbm, dst_hbm, zeros_hbm, h0_hbm, parts_hbm,
          ewb_a, ewb_b, p_a, p_b, r_a, r_b, is_a, is_b, id_a, id_b, acc,
          s_ea, s_eb, s_pa, s_pb, s_ia, s_ib, s_da, s_db,
          s_sta, s_stb, s_sca, s_scb):
        cid = lax.axis_index("c")
        sid = lax.axis_index("s")
        wid = sid * NC + cid
        base = wid * b_per_w

        A = (ewb_a, p_a, r_a, is_a, id_a, s_ea, s_pa, s_ia, s_da, s_sta, s_sca)
        B = (ewb_b, p_b, r_b, is_b, id_b, s_eb, s_pb, s_ib, s_db, s_stb, s_scb)

        def off(c):
            return pl.multiple_of(base + c * chunk, 8)

        def ld_ewb(c, S):
            return _mac(ewb_hbm.at[pl.ds(off(c), chunk)], S[0], S[5])

        def ld_is(c, S):
            return _mac(src_hbm.at[pl.ds(off(c), chunk)], S[3], S[7])

        def ld_id(c, S):
            return _mac(dst_hbm.at[pl.ds(off(c), chunk)], S[4], S[8])

        def gat(S):
            return _mac(p_hbm.at[S[3]], S[1], S[6])

        def st(c, S):
            return _mac(S[0], h0_hbm.at[pl.ds(off(c), chunk)], S[9])

        def sca(S):
            return _mac(S[2], acc.at[S[4]], S[10])

        def compute(S):
            ewb_v, p_v, r_v = S[0], S[1], S[2]

            def row(r, carry):
                for g in range(dm // 16):
                    sl = pl.ds(g * 16, 16)
                    v = ewb_v[r, sl] + p_v[r, sl]
                    ewb_v[r, sl] = v
                    r_v[r, sl] = jnp.maximum(v, 0.0)
                return carry

            lax.fori_loop(0, chunk, row, 0)

        def start_loads(c, S):
            ld_ewb(c, S).start()
            ld_is(c, S).start()
            ld_id(c, S).start()

        @pl.when(sid == 0)
        def _():
            pltpu.sync_copy(zeros_hbm, acc)

        plsc.subcore_barrier()
        start_loads(0, A)

        def body(j, carry):
            c0 = 2 * j
            c1 = c0 + 1
            ld_is(c0, A).wait()
            gat(A).start()

            @pl.when(j >= 1)
            def _():
                st(c1 - 2, B).wait()
                sca(B).wait()

            start_loads(c1, B)
            ld_ewb(c0, A).wait()
            gat(A).wait()
            compute(A)
            ld_id(c0, A).wait()
            st(c0, A).start()
            sca(A).start(add=True)
            ld_is(c1, B).wait()
            gat(B).start()
            ld_ewb(c1, B).wait()
            gat(B).wait()
            compute(B)

            @pl.when(j + 1 < npair)
            def _():
                st(c0, A).wait()
                sca(A).wait()
                start_loads(c0 + 2, A)

            ld_id(c1, B).wait()
            st(c1, B).start()
            sca(B).start(add=True)
            return carry

        lax.fori_loop(0, npair, body, 0)
        st(nch - 2, A).wait()
        sca(A).wait()
        st(nch - 1, B).wait()
        sca(B).wait()
        plsc.subcore_barrier()

        @pl.when(sid == 0)
        def _():
            pltpu.sync_copy(acc, parts_hbm.at[cid])

    return k(ewb, p, src, dst, zeros)


def _sc_ep(h0, tcomb, rev, src2, dst, zeros, write_ht, chunk=40):
    e, dm = h0.shape
    n = zeros.shape[0]
    b_per_w = e // NW
    nch = b_per_w // chunk
    npair = nch // 2
    assert nch % 2 == 0

    buf = lambda: pltpu.VMEM((chunk, dm), jnp.float32)
    ibuf = lambda: pltpu.VMEM((chunk,), jnp.int32)
    sem = pltpu.SemaphoreType.DMA

    outs = [jax.ShapeDtypeStruct((NC, n, dm), jnp.float32)]
    if write_ht:
        outs = [jax.ShapeDtypeStruct((e, dm), jnp.float32)] + outs

    @functools.partial(
        pl.kernel,
        mesh=_sc_mesh(),
        out_type=tuple(outs),
        scratch_types=(
            [buf() for _ in range(6)]
            + [ibuf() for _ in range(6)]
            + [pltpu.VMEM_SHARED((n, dm), jnp.float32)]
            + [sem] * 16
        ),
    )
    def k(h0_hbm, t_hbm, rev_hbm, src_hbm, dst_hbm, zeros_hbm, *rest):
        if write_ht:
            ht_hbm, parts_hbm = rest[0], rest[1]
            rest = rest[2:]
        else:
            parts_hbm = rest[0]
            ht_hbm = None
            rest = rest[1:]
        (h0_a, h0_b, tr_a, tr_b, ts_a, ts_b,
         ir_a, ir_b, is_a, is_b, id_a, id_b, acc,
         s_ha, s_hb, s_ra, s_rb, s_sa, s_sb,
         s_ira, s_irb, s_isa, s_isb, s_ida, s_idb,
         s_sta, s_stb, s_sca, s_scb) = rest

        cid = lax.axis_index("c")
        sid = lax.axis_index("s")
        wid = sid * NC + cid
        base = wid * b_per_w

        A = (h0_a, tr_a, ts_a, ir_a, is_a, id_a,
             s_ha, s_ra, s_sa, s_ira, s_isa, s_ida, s_sta, s_sca)
        B = (h0_b, tr_b, ts_b, ir_b, is_b, id_b,
             s_hb, s_rb, s_sb, s_irb, s_isb, s_idb, s_stb, s_scb)

        def off(c):
            return pl.multiple_of(base + c * chunk, 8)

        def ld_h0(c, S):
            return _mac(h0_hbm.at[pl.ds(off(c), chunk)], S[0], S[6])

        def ld_ir(c, S):
            return _mac(rev_hbm.at[pl.ds(off(c), chunk)], S[3], S[9])

        def ld_is(c, S):
            return _mac(src_hbm.at[pl.ds(off(c), chunk)], S[4], S[10])

        def ld_id(c, S):
            return _mac(dst_hbm.at[pl.ds(off(c), chunk)], S[5], S[11])

        def gat_r(S):
            return _mac(t_hbm.at[S[3]], S[1], S[7])

        def gat_s(S):
            return _mac(t_hbm.at[S[4]], S[2], S[8])

        def st(c, S):
            return _mac(S[0], ht_hbm.at[pl.ds(off(c), chunk)], S[12])

        def sca(S):
            return _mac(S[0], acc.at[S[5]], S[13])

        def compute(S):
            h0_v, tr_v, ts_v = S[0], S[1], S[2]

            def row(r, carry):
                for g in range(dm // 16):
                    sl = pl.ds(g * 16, 16)
                    v = h0_v[r, sl] + ts_v[r, sl] - tr_v[r, sl]
                    h0_v[r, sl] = jnp.maximum(v, 0.0)
                return carry

            lax.fori_loop(0, chunk, row, 0)

        def start_loads(c, S):
            ld_h0(c, S).start()
            ld_ir(c, S).start()
            ld_is(c, S).start()
            ld_id(c, S).start()

        @pl.when(sid == 0)
        def _():
            pltpu.sync_copy(zeros_hbm, acc)

        plsc.subcore_barrier()
        start_loads(0, A)

        def body(j, carry):
            c0 = 2 * j
            c1 = c0 + 1
            ld_ir(c0, A).wait()
            ld_is(c0, A).wait()
            gat_r(A).start()
            gat_s(A).start()

            @pl.when(j >= 1)
            def _():
                if write_ht:
                    st(c1 - 2, B).wait()
                sca(B).wait()

            start_loads(c1, B)
            ld_h0(c0, A).wait()
            gat_r(A).wait()
            gat_s(A).wait()
            compute(A)
            ld_id(c0, A).wait()
            if write_ht:
                st(c0, A).start()
            sca(A).start(add=True)
            ld_ir(c1, B).wait()
            ld_is(c1, B).wait()
            gat_r(B).start()
            gat_s(B).start()
            ld_h0(c1, B).wait()
            gat_r(B).wait()
            gat_s(B).wait()
            compute(B)

            @pl.when(j + 1 < npair)
            def _():
                if write_ht:
                    st(c0, A).wait()
                sca(A).wait()
                start_loads(c0 + 2, A)

            ld_id(c1, B).wait()
            if write_ht:
                st(c1, B).start()
            sca(B).start(add=True)
            return carry

        lax.fori_loop(0, npair, body, 0)
        if write_ht:
            st(nch - 2, A).wait()
            st(nch - 1, B).wait()
        sca(A).wait()
        sca(B).wait()
        plsc.subcore_barrier()

        @pl.when(sid == 0)
        def _():
            pltpu.sync_copy(acc, parts_hbm.at[cid])

    return k(h0, tcomb, rev, src2, dst, zeros)


def _tc_pa(x, ea, wx_t, we_t, b_i, bk=400):
    n, d = x.shape
    e, de = ea.shape
    ge = e // bk
    gn = n // bk

    def body(x_ref, ea_ref, wx_ref, we_ref, b_ref, ewb_ref, p_ref):
        i = pl.program_id(0)

        @pl.when(i < ge)
        def _():
            ewb_ref[...] = jnp.dot(ea_ref[...], we_ref[...],
                                   preferred_element_type=jnp.float32) + b_ref[...]

        @pl.when(i >= ge)
        def _():
            p_ref[...] = jnp.dot(x_ref[...], wx_ref[...],
                                 preferred_element_type=jnp.float32)

    return pl.pallas_call(
        body,
        grid=(ge + gn,),
        in_specs=[
            pl.BlockSpec((bk, d), lambda i: (jnp.maximum(i - ge, 0), 0)),
            pl.BlockSpec((bk, de), lambda i: (jnp.minimum(i, ge - 1), 0)),
            pl.BlockSpec((d, 128), lambda i: (0, 0)),
            pl.BlockSpec((de, 128), lambda i: (0, 0)),
            pl.BlockSpec((1, 128), lambda i: (0, 0)),
        ],
        out_specs=[
            pl.BlockSpec((bk, 128), lambda i: (jnp.minimum(i, ge - 1), 0)),
            pl.BlockSpec((bk, 128), lambda i: (jnp.maximum(i - ge, 0), 0)),
        ],
        out_shape=[jax.ShapeDtypeStruct((e, 128), jnp.float32),
                   jax.ShapeDtypeStruct((n, 128), jnp.float32)],
        compiler_params=pltpu.CompilerParams(dimension_semantics=("arbitrary",)),
    )(x, ea, wx_t, we_t, b_i)


def _tc_t(h, p0, p1, wh_t, b_h, relu_in, bk=400):
    e = h.shape[0]
    n = p0.shape[0]
    ge = e // bk
    gn = n // bk

    def body(h_ref, p0_ref, p1_ref, w_ref, b_ref, t_ref):
        i = pl.program_id(0)

        @pl.when(i < ge)
        def _():
            hv = h_ref[...]
            if relu_in:
                hv = jnp.maximum(hv, 0.0)
            t_ref[...] = jnp.dot(hv, w_ref[...],
                                 preferred_element_type=jnp.float32)

        @pl.when(i >= ge)
        def _():
            t_ref[...] = jnp.dot(p0_ref[...] + p1_ref[...], w_ref[...],
                                 preferred_element_type=jnp.float32) + b_ref[...]

    return pl.pallas_call(
        body,
        grid=(ge + gn,),
        in_specs=[
            pl.BlockSpec((bk, 128), lambda i: (jnp.minimum(i, ge - 1), 0)),
            pl.BlockSpec((bk, 128), lambda i: (jnp.maximum(i - ge, 0), 0)),
            pl.BlockSpec((bk, 128), lambda i: (jnp.maximum(i - ge, 0), 0)),
            pl.BlockSpec((128, 128), lambda i: (0, 0)),
            pl.BlockSpec((1, 128), lambda i: (0, 0)),
        ],
        out_specs=pl.BlockSpec((bk, 128), lambda i: (i, 0)),
        out_shape=jax.ShapeDtypeStruct((e + n, 128), jnp.float32),
        compiler_params=pltpu.CompilerParams(dimension_semantics=("arbitrary",)),
    )(h, p0, p1, wh_t, b_h)


def _tc_out(x, p0, p1, wox_t, wom_t, b_o, bn=1000):
    n = x.shape[0]

    def body(x_ref, p0_ref, p1_ref, wx_ref, wm_ref, b_ref, o_ref):
        agg = p0_ref[...] + p1_ref[...]
        xv = x_ref[...]
        m = jnp.where(jnp.sum(agg, axis=1, keepdims=True) == 0.0, xv, agg)
        acc = jnp.dot(xv, wx_ref[...], preferred_element_type=jnp.float32)
        acc += jnp.dot(m, wm_ref[...], preferred_element_type=jnp.float32)
        o_ref[...] = jnp.maximum(acc + b_ref[...], 0.0)

    return pl.pallas_call(
        body,
        grid=(n // bn,),
        in_specs=[pl.BlockSpec((bn, 128), lambda i: (i, 0)),
                  pl.BlockSpec((bn, 128), lambda i: (i, 0)),
                  pl.BlockSpec((bn, 128), lambda i: (i, 0)),
                  pl.BlockSpec((128, 128), lambda i: (0, 0)),
                  pl.BlockSpec((128, 128), lambda i: (0, 0)),
                  pl.BlockSpec((1, 128), lambda i: (0, 0))],
        out_specs=pl.BlockSpec((bn, 128), lambda i: (i, 0)),
        out_shape=jax.ShapeDtypeStruct((n, 128), jnp.float32),
        compiler_params=pltpu.CompilerParams(dimension_semantics=("parallel",)),
    )(x, p0, p1, wox_t, wom_t, b_o)


def kernel(x, edge_attr, W_i, b_i, W_h, b_h, W_o, b_o, edge_index, rev_edge_index):
    n, d = x.shape
    e = edge_attr.shape[0]
    hid = W_h.shape[0]
    src = edge_index[0]
    dst = edge_index[1]
    src2 = src + jnp.int32(e)

    wx_t = W_i[:, :d].T
    we_t = W_i[:, d:].T
    wh_t = W_h.T
    wox_t = W_o[:, :d].T
    wom_t = W_o[:, d:].T
    b_i2 = b_i.reshape(1, hid)
    b_h2 = b_h.reshape(1, hid)
    b_o2 = b_o.reshape(1, hid)
    zeros_n = jnp.zeros((n, hid), jnp.float32)

    ewb, p = _tc_pa(x, edge_attr, wx_t, we_t, b_i2)
    h0, parts = _sc_ep_a(ewb, p, src, dst, zeros_n)

    t1 = _tc_t(h0, parts[0], parts[1], wh_t, b_h2, relu_in=True)
    ht, parts = _sc_ep(h0, t1, rev_edge_index, src2, dst, zeros_n, write_ht=True)

    t2 = _tc_t(ht, parts[0], parts[1], wh_t, b_h2, relu_in=False)
    parts, = _sc_ep(h0, t2, rev_edge_index, src2, dst, zeros_n, write_ht=False)

    return _tc_out(x, parts[0], parts[1], wox_t, wom_t, b_o2)


# speedup vs baseline: 1.3193x; 1.3193x over previous
import functools

import jax
import jax.numpy as jnp
from jax import lax
from jax.experimental import pallas as pl
from jax.experimental.pallas import tpu as pltpu
from jax.experimental.pallas import tpu_sc as plsc

NC = 2
NS = 16
NW = NC * NS


def _sc_mesh():
    return plsc.VectorSubcoreMesh(
        core_axis_name="c", subcore_axis_name="s", num_cores=NC, num_subcores=NS
    )


def _mac(src, dst, sem):
    return pltpu.make_async_copy(src, dst, sem)


def _sc_ep_a(ewb, p, src, dst, zeros, chunk=40):
    e, dm = ewb.shape
    n = zeros.shape[0]
    b_per_w = e // NW
    nch = b_per_w // chunk
    npair = nch // 2
    assert nch % 2 == 0

    buf = lambda: pltpu.VMEM((chunk, dm), jnp.float32)
    ibuf = lambda: pltpu.VMEM((chunk,), jnp.int32)
    sem = pltpu.SemaphoreType.DMA

    @functools.partial(
        pl.kernel,
        mesh=_sc_mesh(),
        out_type=(jax.ShapeDtypeStruct((e, dm), jnp.float32),
                  jax.ShapeDtypeStruct((NC, n, dm), jnp.float32)),
        scratch_types=(
            [buf() for _ in range(4)]
            + [buf() for _ in range(2)]
            + [ibuf() for _ in range(4)]
            + [pltpu.VMEM_SHARED((n, dm), jnp.float32)]
            + [sem] * 12
        ),
    )
    def k(ewb_hbm, p_hbm, src_hbm, dst_hbm, zeros_hbm, h0_hbm, parts_hbm,
          ewb_a, ewb_b, p_a, p_b, r_a, r_b, is_a, is_b, id_a, id_b, acc,
          s_ea, s_eb, s_pa, s_pb, s_ia, s_ib, s_da, s_db,
          s_sta, s_stb, s_sca, s_scb):
        cid = lax.axis_index("c")
        sid = lax.axis_index("s")
        wid = sid * NC + cid
        base = wid * b_per_w

        A = (ewb_a, p_a, r_a, is_a, id_a, s_ea, s_pa, s_ia, s_da, s_sta, s_sca)
        B = (ewb_b, p_b, r_b, is_b, id_b, s_eb, s_pb, s_ib, s_db, s_stb, s_scb)

        def off(c):
            return pl.multiple_of(base + c * chunk, 8)

        def ld_ewb(c, S):
            return _mac(ewb_hbm.at[pl.ds(off(c), chunk)], S[0], S[5])

        def ld_is(c, S):
            return _mac(src_hbm.at[pl.ds(off(c), chunk)], S[3], S[7])

        def ld_id(c, S):
            return _mac(dst_hbm.at[pl.ds(off(c), chunk)], S[4], S[8])

        def gat(S):
            return _mac(p_hbm.at[S[3]], S[1], S[6])

        def st(c, S):
            return _mac(S[0], h0_hbm.at[pl.ds(off(c), chunk)], S[9])

        def sca(S):
            return _mac(S[2], acc.at[S[4]], S[10])

        def compute(S):
            ewb_v, p_v, r_v = S[0], S[1], S[2]

            def row(r, carry):
                for g in range(dm // 16):
                    sl = pl.ds(g * 16, 16)
                    v = ewb_v[r, sl] + p_v[r, sl]
                    ewb_v[r, sl] = v
                    r_v[r, sl] = jnp.maximum(v, 0.0)
                return carry

            lax.fori_loop(0, chunk, row, 0)

        def start_loads(c, S):
            ld_ewb(c, S).start()
            ld_is(c, S).start()
            ld_id(c, S).start()

        @pl.when(sid == 0)
        def _():
            pltpu.sync_copy(zeros_hbm, acc)

        plsc.subcore_barrier()
        start_loads(0, A)

        def body(j, carry):
            c0 = 2 * j
            c1 = c0 + 1
            ld_is(c0, A).wait()
            gat(A).start()

            @pl.when(j >= 1)
            def _():
                st(c1 - 2, B).wait()
                sca(B).wait()

            start_loads(c1, B)
            ld_ewb(c0, A).wait()
            gat(A).wait()
            compute(A)
            ld_id(c0, A).wait()
            st(c0, A).start()
            sca(A).start(add=True)
            ld_is(c1, B).wait()
            gat(B).start()
            ld_ewb(c1, B).wait()
            gat(B).wait()
            compute(B)

            @pl.when(j + 1 < npair)
            def _():
                st(c0, A).wait()
                sca(A).wait()
                start_loads(c0 + 2, A)

            ld_id(c1, B).wait()
            st(c1, B).start()
            sca(B).start(add=True)
            return carry

        lax.fori_loop(0, npair, body, 0)
        st(nch - 2, A).wait()
        sca(A).wait()
        st(nch - 1, B).wait()
        sca(B).wait()
        plsc.subcore_barrier()

        @pl.when(sid == 0)
        def _():
            pltpu.sync_copy(acc, parts_hbm.at[cid])

    return k(ewb, p, src, dst, zeros)


def _sc_ep(h0, tcomb, rev, src2, dst, zeros, write_ht, chunk=40):
    e, dm = h0.shape
    n = zeros.shape[0]
    b_per_w = e // NW
    nch = b_per_w // chunk
    npair = nch // 2
    assert nch % 2 == 0

    buf = lambda: pltpu.VMEM((chunk, dm), jnp.float32)
    ibuf = lambda: pltpu.VMEM((chunk,), jnp.int32)
    sem = pltpu.SemaphoreType.DMA

    outs = [jax.ShapeDtypeStruct((NC, n, dm), jnp.float32)]
    if write_ht:
        outs = [jax.ShapeDtypeStruct((e, dm), jnp.float32)] + outs

    @functools.partial(
        pl.kernel,
        mesh=_sc_mesh(),
        out_type=tuple(outs),
        scratch_types=(
            [buf() for _ in range(6)]
            + [ibuf() for _ in range(6)]
            + [pltpu.VMEM_SHARED((n, dm), jnp.float32)]
            + [sem] * 16
        ),
    )
    def k(h0_hbm, t_hbm, rev_hbm, src_hbm, dst_hbm, zeros_hbm, *rest):
        if write_ht:
            ht_hbm, parts_hbm = rest[0], rest[1]
            rest = rest[2:]
        else:
            parts_hbm = rest[0]
            ht_hbm = None
            rest = rest[1:]
        (h0_a, h0_b, tr_a, tr_b, ts_a, ts_b,
         ir_a, ir_b, is_a, is_b, id_a, id_b, acc,
         s_ha, s_hb, s_ra, s_rb, s_sa, s_sb,
         s_ira, s_irb, s_isa, s_isb, s_ida, s_idb,
         s_sta, s_stb, s_sca, s_scb) = rest

        cid = lax.axis_index("c")
        sid = lax.axis_index("s")
        wid = sid * NC + cid
        base = wid * b_per_w

        A = (h0_a, tr_a, ts_a, ir_a, is_a, id_a,
             s_ha, s_ra, s_sa, s_ira, s_isa, s_ida, s_sta, s_sca)
        B = (h0_b, tr_b, ts_b, ir_b, is_b, id_b,
             s_hb, s_rb, s_sb, s_irb, s_isb, s_idb, s_stb, s_scb)

        def off(c):
            return pl.multiple_of(base + c * chunk, 8)

        def ld_h0(c, S):
            return _mac(h0_hbm.at[pl.ds(off(c), chunk)], S[0], S[6])

        def ld_ir(c, S):
            return _mac(rev_hbm.at[pl.ds(off(c), chunk)], S[3], S[9])

        def ld_is(c, S):
            return _mac(src_hbm.at[pl.ds(off(c), chunk)], S[4], S[10])

        def ld_id(c, S):
            return _mac(dst_hbm.at[pl.ds(off(c), chunk)], S[5], S[11])

        def gat_r(S):
            return _mac(t_hbm.at[S[3]], S[1], S[7])

        def gat_s(S):
            return _mac(t_hbm.at[S[4]], S[2], S[8])

        def st(c, S):
            return _mac(S[0], ht_hbm.at[pl.ds(off(c), chunk)], S[12])

        def sca(S):
            return _mac(S[0], acc.at[S[5]], S[13])

        def compute(S):
            h0_v, tr_v, ts_v = S[0], S[1], S[2]

            def row(r, carry):
                for g in range(dm // 16):
                    sl = pl.ds(g * 16, 16)
                    v = h0_v[r, sl] + ts_v[r, sl] - tr_v[r, sl]
                    h0_v[r, sl] = jnp.maximum(v, 0.0)
                return carry

            lax.fori_loop(0, chunk, row, 0)

        def start_loads(c, S):
            ld_h0(c, S).start()
            ld_ir(c, S).start()
            ld_is(c, S).start()
            ld_id(c, S).start()

        @pl.when(sid == 0)
        def _():
            pltpu.sync_copy(zeros_hbm, acc)

        plsc.subcore_barrier()
        start_loads(0, A)

        def body(j, carry):
            c0 = 2 * j
            c1 = c0 + 1
            ld_ir(c0, A).wait()
            ld_is(c0, A).wait()
            gat_r(A).start()
            gat_s(A).start()

            @pl.when(j >= 1)
            def _():
                if write_ht:
                    st(c1 - 2, B).wait()
                sca(B).wait()

            start_loads(c1, B)
            ld_h0(c0, A).wait()
            gat_r(A).wait()
            gat_s(A).wait()
            compute(A)
            ld_id(c0, A).wait()
            if write_ht:
                st(c0, A).start()
            sca(A).start(add=True)
            ld_ir(c1, B).wait()
            ld_is(c1, B).wait()
            gat_r(B).start()
            gat_s(B).start()
            ld_h0(c1, B).wait()
            gat_r(B).wait()
            gat_s(B).wait()
            compute(B)

            @pl.when(j + 1 < npair)
            def _():
                if write_ht:
                    st(c0, A).wait()
                sca(A).wait()
                start_loads(c0 + 2, A)

            ld_id(c1, B).wait()
            if write_ht:
                st(c1, B).start()
            sca(B).start(add=True)
            return carry

        lax.fori_loop(0, npair, body, 0)
        if write_ht:
            st(nch - 2, A).wait()
            st(nch - 1, B).wait()
        sca(A).wait()
        sca(B).wait()
        plsc.subcore_barrier()

        @pl.when(sid == 0)
        def _():
            pltpu.sync_copy(acc, parts_hbm.at[cid])

    return k(h0, tcomb, rev, src2, dst, zeros)


def _tc_pa(x, ea, wx_t, we_t, b_i, bk=1000):
    n, d = x.shape
    e, de = ea.shape
    ge = e // bk
    gn = n // bk

    def body(x_ref, ea_ref, wx_ref, we_ref, b_ref, ewb_ref, p_ref):
        i = pl.program_id(0)

        @pl.when(i < ge)
        def _():
            ewb_ref[...] = jnp.dot(ea_ref[...], we_ref[...],
                                   preferred_element_type=jnp.float32) + b_ref[...]

        @pl.when(i >= ge)
        def _():
            p_ref[...] = jnp.dot(x_ref[...], wx_ref[...],
                                 preferred_element_type=jnp.float32)

    return pl.pallas_call(
        body,
        grid=(ge + gn,),
        in_specs=[
            pl.BlockSpec((bk, d), lambda i: (jnp.maximum(i - ge, 0), 0)),
            pl.BlockSpec((bk, de), lambda i: (jnp.minimum(i, ge - 1), 0)),
            pl.BlockSpec((d, 128), lambda i: (0, 0)),
            pl.BlockSpec((de, 128), lambda i: (0, 0)),
            pl.BlockSpec((1, 128), lambda i: (0, 0)),
        ],
        out_specs=[
            pl.BlockSpec((bk, 128), lambda i: (jnp.minimum(i, ge - 1), 0)),
            pl.BlockSpec((bk, 128), lambda i: (jnp.maximum(i - ge, 0), 0)),
        ],
        out_shape=[jax.ShapeDtypeStruct((e, 128), jnp.float32),
                   jax.ShapeDtypeStruct((n, 128), jnp.float32)],
        compiler_params=pltpu.CompilerParams(dimension_semantics=("arbitrary",)),
    )(x, ea, wx_t, we_t, b_i)


def _tc_t(h, p0, p1, wh_t, b_h, relu_in, bk=1000):
    e = h.shape[0]
    n = p0.shape[0]
    ge = e // bk
    gn = n // bk

    def body(h_ref, p0_ref, p1_ref, w_ref, b_ref, t_ref):
        i = pl.program_id(0)

        @pl.when(i < ge)
        def _():
            hv = h_ref[...]
            if relu_in:
                hv = jnp.maximum(hv, 0.0)
            t_ref[...] = jnp.dot(hv, w_ref[...],
                                 preferred_element_type=jnp.float32)

        @pl.when(i >= ge)
        def _():
            t_ref[...] = jnp.dot(p0_ref[...] + p1_ref[...], w_ref[...],
                                 preferred_element_type=jnp.float32) + b_ref[...]

    return pl.pallas_call(
        body,
        grid=(ge + gn,),
        in_specs=[
            pl.BlockSpec((bk, 128), lambda i: (jnp.minimum(i, ge - 1), 0)),
            pl.BlockSpec((bk, 128), lambda i: (jnp.maximum(i - ge, 0), 0)),
            pl.BlockSpec((bk, 128), lambda i: (jnp.maximum(i - ge, 0), 0)),
            pl.BlockSpec((128, 128), lambda i: (0, 0)),
            pl.BlockSpec((1, 128), lambda i: (0, 0)),
        ],
        out_specs=pl.BlockSpec((bk, 128), lambda i: (i, 0)),
        out_shape=jax.ShapeDtypeStruct((e + n, 128), jnp.float32),
        compiler_params=pltpu.CompilerParams(dimension_semantics=("arbitrary",)),
    )(h, p0, p1, wh_t, b_h)


def _tc_out(x, p0, p1, wox_t, wom_t, b_o, bn=1000):
    n = x.shape[0]

    def body(x_ref, p0_ref, p1_ref, wx_ref, wm_ref, b_ref, o_ref):
        agg = p0_ref[...] + p1_ref[...]
        xv = x_ref[...]
        m = jnp.where(jnp.sum(agg, axis=1, keepdims=True) == 0.0, xv, agg)
        acc = jnp.dot(xv, wx_ref[...], preferred_element_type=jnp.float32)
        acc += jnp.dot(m, wm_ref[...], preferred_element_type=jnp.float32)
        o_ref[...] = jnp.maximum(acc + b_ref[...], 0.0)

    return pl.pallas_call(
        body,
        grid=(n // bn,),
        in_specs=[pl.BlockSpec((bn, 128), lambda i: (i, 0)),
                  pl.BlockSpec((bn, 128), lambda i: (i, 0)),
                  pl.BlockSpec((bn, 128), lambda i: (i, 0)),
                  pl.BlockSpec((128, 128), lambda i: (0, 0)),
                  pl.BlockSpec((128, 128), lambda i: (0, 0)),
                  pl.BlockSpec((1, 128), lambda i: (0, 0))],
        out_specs=pl.BlockSpec((bn, 128), lambda i: (i, 0)),
        out_shape=jax.ShapeDtypeStruct((n, 128), jnp.float32),
        compiler_params=pltpu.CompilerParams(dimension_semantics=("parallel",)),
    )(x, p0, p1, wox_t, wom_t, b_o)


def kernel(x, edge_attr, W_i, b_i, W_h, b_h, W_o, b_o, edge_index, rev_edge_index):
    n, d = x.shape
    e = edge_attr.shape[0]
    hid = W_h.shape[0]
    src = edge_index[0]
    dst = edge_index[1]
    src2 = src + jnp.int32(e)

    wx_t = W_i[:, :d].T
    we_t = W_i[:, d:].T
    wh_t = W_h.T
    wox_t = W_o[:, :d].T
    wom_t = W_o[:, d:].T
    b_i2 = b_i.reshape(1, hid)
    b_h2 = b_h.reshape(1, hid)
    b_o2 = b_o.reshape(1, hid)
    zeros_n = jnp.zeros((n, hid), jnp.float32)

    ewb, p = _tc_pa(x, edge_attr, wx_t, we_t, b_i2)
    h0, parts = _sc_ep_a(ewb, p, src, dst, zeros_n)

    t1 = _tc_t(h0, parts[0], parts[1], wh_t, b_h2, relu_in=True)
    ht, parts = _sc_ep(h0, t1, rev_edge_index, src2, dst, zeros_n, write_ht=True)

    t2 = _tc_t(ht, parts[0], parts[1], wh_t, b_h2, relu_in=False)
    parts, = _sc_ep(h0, t2, rev_edge_index, src2, dst, zeros_n, write_ht=False)

    return _tc_out(x, parts[0], parts[1], wox_t, wom_t, b_o2)


# speedup vs baseline: 1.4025x; 1.0630x over previous
import functools

import jax
import jax.numpy as jnp
from jax import lax
from jax.experimental import pallas as pl
from jax.experimental.pallas import tpu as pltpu
from jax.experimental.pallas import tpu_sc as plsc

NC = 2
NS = 16
NW = NC * NS


def _sc_mesh():
    return plsc.VectorSubcoreMesh(
        core_axis_name="c", subcore_axis_name="s", num_cores=NC, num_subcores=NS
    )


def _mac(src, dst, sem):
    return pltpu.make_async_copy(src, dst, sem)


def _sc_ep_a(ewb, p, src, dst, zeros, chunk=40):
    e, dm = ewb.shape
    n = zeros.shape[0]
    b_per_w = e // NW
    nch = b_per_w // chunk
    npair = nch // 2
    assert nch % 2 == 0

    buf = lambda: pltpu.VMEM((chunk, dm), jnp.float32)
    ibuf = lambda: pltpu.VMEM((chunk,), jnp.int32)
    sem = pltpu.SemaphoreType.DMA

    @functools.partial(
        pl.kernel,
        mesh=_sc_mesh(),
        out_type=(jax.ShapeDtypeStruct((e, dm), jnp.float32),
                  jax.ShapeDtypeStruct((NC, n, dm), jnp.float32)),
        scratch_types=(
            [buf() for _ in range(4)]
            + [buf() for _ in range(2)]
            + [ibuf() for _ in range(4)]
            + [pltpu.VMEM_SHARED((n, dm), jnp.float32)]
            + [sem] * 12
        ),
    )
    def k(ewb_hbm, p_hbm, src_hbm, dst_hbm, zeros_hbm, h0_hbm, parts_hbm,
          ewb_a, ewb_b, p_a, p_b, r_a, r_b, is_a, is_b, id_a, id_b, acc,
          s_ea, s_eb, s_pa, s_pb, s_ia, s_ib, s_da, s_db,
          s_sta, s_stb, s_sca, s_scb):
        cid = lax.axis_index("c")
        sid = lax.axis_index("s")
        wid = sid * NC + cid
        base = wid * b_per_w

        A = (ewb_a, p_a, r_a, is_a, id_a, s_ea, s_pa, s_ia, s_da, s_sta, s_sca)
        B = (ewb_b, p_b, r_b, is_b, id_b, s_eb, s_pb, s_ib, s_db, s_stb, s_scb)

        def off(c):
            return pl.multiple_of(base + c * chunk, 8)

        def ld_ewb(c, S):
            return _mac(ewb_hbm.at[pl.ds(off(c), chunk)], S[0], S[5])

        def ld_is(c, S):
            return _mac(src_hbm.at[pl.ds(off(c), chunk)], S[3], S[7])

        def ld_id(c, S):
            return _mac(dst_hbm.at[pl.ds(off(c), chunk)], S[4], S[8])

        def gat(S):
            return _mac(p_hbm.at[S[3]], S[1], S[6])

        def st(c, S):
            return _mac(S[0], h0_hbm.at[pl.ds(off(c), chunk)], S[9])

        def sca(S):
            return _mac(S[2], acc.at[S[4]], S[10])

        def compute(S):
            ewb_v, p_v, r_v = S[0], S[1], S[2]

            def row(r, carry):
                for g in range(dm // 16):
                    sl = pl.ds(g * 16, 16)
                    v = ewb_v[r, sl] + p_v[r, sl]
                    ewb_v[r, sl] = v
                    r_v[r, sl] = jnp.maximum(v, 0.0)
                return carry

            lax.fori_loop(0, chunk, row, 0)

        def start_loads(c, S):
            ld_ewb(c, S).start()
            ld_is(c, S).start()
            ld_id(c, S).start()

        @pl.when(sid == 0)
        def _():
            pltpu.sync_copy(zeros_hbm, acc)

        plsc.subcore_barrier()
        start_loads(0, A)

        def body(j, carry):
            c0 = 2 * j
            c1 = c0 + 1
            ld_is(c0, A).wait()
            gat(A).start()

            @pl.when(j >= 1)
            def _():
                st(c1 - 2, B).wait()
                sca(B).wait()

            start_loads(c1, B)
            ld_ewb(c0, A).wait()
            gat(A).wait()
            compute(A)
            ld_id(c0, A).wait()
            st(c0, A).start()
            sca(A).start(add=True)
            ld_is(c1, B).wait()
            gat(B).start()
            ld_ewb(c1, B).wait()
            gat(B).wait()
            compute(B)

            @pl.when(j + 1 < npair)
            def _():
                st(c0, A).wait()
                sca(A).wait()
                start_loads(c0 + 2, A)

            ld_id(c1, B).wait()
            st(c1, B).start()
            sca(B).start(add=True)
            return carry

        lax.fori_loop(0, npair, body, 0)
        st(nch - 2, A).wait()
        sca(A).wait()
        st(nch - 1, B).wait()
        sca(B).wait()
        plsc.subcore_barrier()

        @pl.when(sid == 0)
        def _():
            pltpu.sync_copy(acc, parts_hbm.at[cid])

    return k(ewb, p, src, dst, zeros)


def _sc_ep(h0, tw, rev, src2, dst, zeros, write_ht, chunk=40):
    e, dm = h0.shape
    n = zeros.shape[0]
    dw = dm // 2
    b_per_w = e // NW
    nch = b_per_w // chunk
    npair = nch // 2
    tail = nch % 2 == 1

    buf = lambda: pltpu.VMEM((chunk, dm), jnp.float32)
    wbuf = lambda: pltpu.VMEM((chunk, dm), jnp.float32)
    ibuf = lambda: pltpu.VMEM((chunk,), jnp.int32)
    sem = pltpu.SemaphoreType.DMA

    outs = [jax.ShapeDtypeStruct((NC, n, dm), jnp.float32)]
    if write_ht:
        outs = [jax.ShapeDtypeStruct((e, dm), jnp.float32)] + outs

    @functools.partial(
        pl.kernel,
        mesh=_sc_mesh(),
        out_type=tuple(outs),
        scratch_types=(
            [buf() for _ in range(2)]
            + [wbuf() for _ in range(4)]
            + [ibuf() for _ in range(6)]
            + [pltpu.VMEM_SHARED((n, dm), jnp.float32)]
            + [sem] * 16
        ),
    )
    def k(h0_hbm, t_hbm, rev_hbm, src_hbm, dst_hbm, zeros_hbm, *rest):
        if write_ht:
            ht_hbm, parts_hbm = rest[0], rest[1]
            rest = rest[2:]
        else:
            parts_hbm = rest[0]
            ht_hbm = None
            rest = rest[1:]
        (h0_a, h0_b, tr_a, tr_b, ts_a, ts_b,
         ir_a, ir_b, is_a, is_b, id_a, id_b, acc,
         s_ha, s_hb, s_ra, s_rb, s_sa, s_sb,
         s_ira, s_irb, s_isa, s_isb, s_ida, s_idb,
         s_sta, s_stb, s_sca, s_scb) = rest

        cid = lax.axis_index("c")
        sid = lax.axis_index("s")
        wid = sid * NC + cid
        base = wid * b_per_w

        A = (h0_a, tr_a, ts_a, ir_a, is_a, id_a,
             s_ha, s_ra, s_sa, s_ira, s_isa, s_ida, s_sta, s_sca)
        B = (h0_b, tr_b, ts_b, ir_b, is_b, id_b,
             s_hb, s_rb, s_sb, s_irb, s_isb, s_idb, s_stb, s_scb)

        def off(c):
            return pl.multiple_of(base + c * chunk, 8)

        def ld_h0(c, S):
            return _mac(h0_hbm.at[pl.ds(off(c), chunk)], S[0], S[6])

        def ld_ir(c, S):
            return _mac(rev_hbm.at[pl.ds(off(c), chunk)], S[3], S[9])

        def ld_is(c, S):
            return _mac(src_hbm.at[pl.ds(off(c), chunk)], S[4], S[10])

        def ld_id(c, S):
            return _mac(dst_hbm.at[pl.ds(off(c), chunk)], S[5], S[11])

        def gat_r(S):
            return _mac(t_hbm.at[S[3]], S[1], S[7])

        def gat_s(S):
            return _mac(t_hbm.at[S[4]], S[2], S[8])

        def st(c, S):
            return _mac(S[0], ht_hbm.at[pl.ds(off(c), chunk)], S[12])

        def sca(S):
            return _mac(S[0], acc.at[S[5]], S[13])

        def compute(S):
            h0_v, tr_v, ts_v = S[0], S[1], S[2]

            def row(r, carry):
                for g in range(dm // 16):
                    sl = pl.ds(g * 16, 16)
                    v = h0_v[r, sl] + ts_v[r, sl] - tr_v[r, sl]
                    h0_v[r, sl] = jnp.maximum(v, 0.0)
                return carry

            lax.fori_loop(0, chunk, row, 0)

        def start_loads(c, S):
            ld_h0(c, S).start()
            ld_ir(c, S).start()
            ld_is(c, S).start()
            ld_id(c, S).start()

        @pl.when(sid == 0)
        def _():
            pltpu.sync_copy(zeros_hbm, acc)

        plsc.subcore_barrier()
        start_loads(0, A)

        def body(j, carry):
            c0 = 2 * j
            c1 = c0 + 1
            ld_ir(c0, A).wait()
            ld_is(c0, A).wait()
            gat_r(A).start()
            gat_s(A).start()

            @pl.when(j >= 1)
            def _():
                if write_ht:
                    st(c1 - 2, B).wait()
                sca(B).wait()

            start_loads(c1, B)
            ld_h0(c0, A).wait()
            gat_r(A).wait()
            gat_s(A).wait()
            compute(A)
            ld_id(c0, A).wait()
            if write_ht:
                st(c0, A).start()
            sca(A).start(add=True)
            ld_ir(c1, B).wait()
            ld_is(c1, B).wait()
            gat_r(B).start()
            gat_s(B).start()
            ld_h0(c1, B).wait()
            gat_r(B).wait()
            gat_s(B).wait()
            compute(B)

            @pl.when(c0 + 2 < nch)
            def _():
                if write_ht:
                    st(c0, A).wait()
                sca(A).wait()
                start_loads(c0 + 2, A)

            ld_id(c1, B).wait()
            if write_ht:
                st(c1, B).start()
            sca(B).start(add=True)
            return carry

        lax.fori_loop(0, npair, body, 0)
        if tail:
            c = nch - 1
            ld_ir(c, A).wait()
            ld_is(c, A).wait()
            gat_r(A).start()
            gat_s(A).start()
            ld_h0(c, A).wait()
            gat_r(A).wait()
            gat_s(A).wait()
            compute(A)
            ld_id(c, A).wait()
            if write_ht:
                st(c, A).start()
            sca(A).start(add=True)
            if write_ht:
                st(c, A).wait()
            sca(A).wait()
        else:
            if write_ht:
                st(nch - 2, A).wait()
            sca(A).wait()
        if write_ht:
            st(nch - 2 if tail else nch - 1, B).wait()
        sca(B).wait()
        plsc.subcore_barrier()

        @pl.when(sid == 0)
        def _():
            pltpu.sync_copy(acc, parts_hbm.at[cid])

    return k(h0, tw, rev, src2, dst, zeros)


def _tc_a(ea_t, we_t, b_i, bk=512):
    de, e = ea_t.shape

    def body(ea_ref, w_ref, b_ref, o_ref):
        o_ref[...] = lax.dot_general(
            ea_ref[...], w_ref[...], (((0,), (0,)), ((), ())),
            preferred_element_type=jnp.float32) + b_ref[...]

    return pl.pallas_call(
        body,
        grid=(e // bk,),
        in_specs=[
            pl.BlockSpec((de, bk), lambda i: (0, i)),
            pl.BlockSpec((de, 128), lambda i: (0, 0)),
            pl.BlockSpec((1, 128), lambda i: (0, 0)),
        ],
        out_specs=pl.BlockSpec((bk, 128), lambda i: (i, 0)),
        out_shape=jax.ShapeDtypeStruct((e, 128), jnp.float32),
        compiler_params=pltpu.CompilerParams(dimension_semantics=("parallel",)),
    )(ea_t, we_t, b_i)


def _tc_p(x, wx_t, bn=1000):
    n, d = x.shape

    def body(x_ref, w_ref, o_ref):
        o_ref[...] = jnp.dot(x_ref[...], w_ref[...],
                             preferred_element_type=jnp.float32)

    return pl.pallas_call(
        body,
        grid=(n // bn,),
        in_specs=[pl.BlockSpec((bn, d), lambda i: (i, 0)),
                  pl.BlockSpec((d, 128), lambda i: (0, 0))],
        out_specs=pl.BlockSpec((bn, 128), lambda i: (i, 0)),
        out_shape=jax.ShapeDtypeStruct((n, 128), jnp.float32),
        compiler_params=pltpu.CompilerParams(dimension_semantics=("parallel",)),
    )(x, wx_t)


def _tc_t(h, p0, p1, wh_t, b_h, relu_in, bk=2000):
    e = h.shape[0]
    n = p0.shape[0]
    ge = e // bk
    gn = n // bk

    def body(h_ref, p0_ref, p1_ref, w_ref, b_ref, t_ref):
        i = pl.program_id(0)

        @pl.when(i < ge)
        def _():
            hv = h_ref[...]
            if relu_in:
                hv = jnp.maximum(hv, 0.0)
            t_ref[...] = jnp.dot(hv, w_ref[...],
                                 preferred_element_type=jnp.float32)

        @pl.when(i >= ge)
        def _():
            t_ref[...] = jnp.dot(p0_ref[...] + p1_ref[...], w_ref[...],
                                 preferred_element_type=jnp.float32) + b_ref[...]

    return pl.pallas_call(
        body,
        grid=(ge + gn,),
        in_specs=[
            pl.BlockSpec((bk, 128), lambda i: (jnp.minimum(i, ge - 1), 0)),
            pl.BlockSpec((bk, 128), lambda i: (jnp.maximum(i - ge, 0), 0)),
            pl.BlockSpec((bk, 128), lambda i: (jnp.maximum(i - ge, 0), 0)),
            pl.BlockSpec((128, 128), lambda i: (0, 0)),
            pl.BlockSpec((1, 128), lambda i: (0, 0)),
        ],
        out_specs=pl.BlockSpec((bk, 128), lambda i: (i, 0)),
        out_shape=jax.ShapeDtypeStruct((e + n, 128), jnp.float32),
        compiler_params=pltpu.CompilerParams(dimension_semantics=("arbitrary",)),
    )(h, p0, p1, wh_t, b_h)


def _tc_out(x, p0, p1, wox_t, wom_t, b_o, bn=1000):
    n = x.shape[0]

    def body(x_ref, p0_ref, p1_ref, wx_ref, wm_ref, b_ref, o_ref):
        agg = p0_ref[...] + p1_ref[...]
        xv = x_ref[...]
        m = jnp.where(jnp.sum(agg, axis=1, keepdims=True) == 0.0, xv, agg)
        acc = jnp.dot(xv, wx_ref[...], preferred_element_type=jnp.float32)
        acc += jnp.dot(m, wm_ref[...], preferred_element_type=jnp.float32)
        o_ref[...] = jnp.maximum(acc + b_ref[...], 0.0)

    return pl.pallas_call(
        body,
        grid=(n // bn,),
        in_specs=[pl.BlockSpec((bn, 128), lambda i: (i, 0)),
                  pl.BlockSpec((bn, 128), lambda i: (i, 0)),
                  pl.BlockSpec((bn, 128), lambda i: (i, 0)),
                  pl.BlockSpec((128, 128), lambda i: (0, 0)),
                  pl.BlockSpec((128, 128), lambda i: (0, 0)),
                  pl.BlockSpec((1, 128), lambda i: (0, 0))],
        out_specs=pl.BlockSpec((bn, 128), lambda i: (i, 0)),
        out_shape=jax.ShapeDtypeStruct((n, 128), jnp.float32),
        compiler_params=pltpu.CompilerParams(dimension_semantics=("parallel",)),
    )(x, p0, p1, wox_t, wom_t, b_o)


def kernel(x, edge_attr, W_i, b_i, W_h, b_h, W_o, b_o, edge_index, rev_edge_index):
    n, d = x.shape
    e = edge_attr.shape[0]
    hid = W_h.shape[0]
    src = edge_index[0]
    dst = edge_index[1]
    src2 = src + jnp.int32(e)

    wx_t = W_i[:, :d].T
    we_t = W_i[:, d:].T
    wh_t = W_h.T
    wox_t = W_o[:, :d].T
    wom_t = W_o[:, d:].T
    b_i2 = b_i.reshape(1, hid)
    b_h2 = b_h.reshape(1, hid)
    b_o2 = b_o.reshape(1, hid)
    zeros_n = jnp.zeros((n, hid), jnp.float32)

    ewb = _tc_a(edge_attr.T, we_t, b_i2)
    p = _tc_p(x, wx_t)
    h0, parts = _sc_ep_a(ewb, p, src, dst, zeros_n)

    t1 = _tc_t(h0, parts[0], parts[1], wh_t, b_h2, relu_in=True)
    ht, parts = _sc_ep(h0, t1, rev_edge_index, src2, dst, zeros_n,
                       write_ht=True)

    t2 = _tc_t(ht, parts[0], parts[1], wh_t, b_h2, relu_in=False)
    parts, = _sc_ep(h0, t2, rev_edge_index, src2, dst, zeros_n,
                    write_ht=False)

    return _tc_out(x, parts[0], parts[1], wox_t, wom_t, b_o2)


# speedup vs baseline: 1.5914x; 1.1347x over previous
import functools

import jax
import jax.numpy as jnp
from jax import lax
from jax.experimental import pallas as pl
from jax.experimental.pallas import tpu as pltpu
from jax.experimental.pallas import tpu_sc as plsc

NC = 2
NS = 16
NW = NC * NS


def _sc_mesh():
    return plsc.VectorSubcoreMesh(
        core_axis_name="c", subcore_axis_name="s", num_cores=NC, num_subcores=NS
    )


def _mac(src, dst, sem):
    return pltpu.make_async_copy(src, dst, sem)


def _sc_ep_a(ewb, p, src, dst, zeros, chunk=40):
    e, dm = ewb.shape
    n = zeros.shape[0]
    b_per_w = e // NW
    nch = b_per_w // chunk
    npair = nch // 2
    assert nch % 2 == 0

    buf = lambda: pltpu.VMEM((chunk, dm), jnp.float32)
    ibuf = lambda: pltpu.VMEM((chunk,), jnp.int32)
    sem = pltpu.SemaphoreType.DMA

    @functools.partial(
        pl.kernel,
        mesh=_sc_mesh(),
        out_type=(jax.ShapeDtypeStruct((e, dm), jnp.float32),
                  jax.ShapeDtypeStruct((NC, n, dm), jnp.float32)),
        scratch_types=(
            [buf() for _ in range(4)]
            + [buf() for _ in range(2)]
            + [ibuf() for _ in range(4)]
            + [pltpu.VMEM_SHARED((n, dm), jnp.float32)]
            + [sem] * 12
        ),
    )
    def k(ewb_hbm, p_hbm, src_hbm, dst_hbm, zeros_hbm, h0_hbm, parts_hbm,
          ewb_a, ewb_b, p_a, p_b, r_a, r_b, is_a, is_b, id_a, id_b, acc,
          s_ea, s_eb, s_pa, s_pb, s_ia, s_ib, s_da, s_db,
          s_sta, s_stb, s_sca, s_scb):
        cid = lax.axis_index("c")
        sid = lax.axis_index("s")
        wid = sid * NC + cid
        base = wid * b_per_w

        A = (ewb_a, p_a, r_a, is_a, id_a, s_ea, s_pa, s_ia, s_da, s_sta, s_sca)
        B = (ewb_b, p_b, r_b, is_b, id_b, s_eb, s_pb, s_ib, s_db, s_stb, s_scb)

        def off(c):
            return pl.multiple_of(base + c * chunk, 8)

        def ld_ewb(c, S):
            return _mac(ewb_hbm.at[pl.ds(off(c), chunk)], S[0], S[5])

        def ld_is(c, S):
            return _mac(src_hbm.at[pl.ds(off(c), chunk)], S[3], S[7])

        def ld_id(c, S):
            return _mac(dst_hbm.at[pl.ds(off(c), chunk)], S[4], S[8])

        def gat(S):
            return _mac(p_hbm.at[S[3]], S[1], S[6])

        def st(c, S):
            return _mac(S[0], h0_hbm.at[pl.ds(off(c), chunk)], S[9])

        def sca(S):
            return _mac(S[2], acc.at[S[4]], S[10])

        def compute(S):
            ewb_v, p_v, r_v = S[0], S[1], S[2]

            def row(r, carry):
                for g in range(dm // 16):
                    sl = pl.ds(g * 16, 16)
                    v = ewb_v[r, sl] + p_v[r, sl]
                    ewb_v[r, sl] = v
                    r_v[r, sl] = jnp.maximum(v, 0.0)
                return carry

            lax.fori_loop(0, chunk, row, 0)

        def start_loads(c, S):
            ld_ewb(c, S).start()
            ld_is(c, S).start()
            ld_id(c, S).start()

        @pl.when(sid == 0)
        def _():
            pltpu.sync_copy(zeros_hbm, acc)

        plsc.subcore_barrier()
        start_loads(0, A)

        def body(j, carry):
            c0 = 2 * j
            c1 = c0 + 1
            ld_is(c0, A).wait()
            gat(A).start()

            @pl.when(j >= 1)
            def _():
                st(c1 - 2, B).wait()
                sca(B).wait()

            start_loads(c1, B)
            ld_ewb(c0, A).wait()
            gat(A).wait()
            compute(A)
            ld_id(c0, A).wait()
            st(c0, A).start()
            sca(A).start(add=True)
            ld_is(c1, B).wait()
            gat(B).start()
            ld_ewb(c1, B).wait()
            gat(B).wait()
            compute(B)

            @pl.when(j + 1 < npair)
            def _():
                st(c0, A).wait()
                sca(A).wait()
                start_loads(c0 + 2, A)

            ld_id(c1, B).wait()
            st(c1, B).start()
            sca(B).start(add=True)
            return carry

        lax.fori_loop(0, npair, body, 0)
        st(nch - 2, A).wait()
        sca(A).wait()
        st(nch - 1, B).wait()
        sca(B).wait()
        plsc.subcore_barrier()

        @pl.when(sid == 0)
        def _():
            pltpu.sync_copy(acc, parts_hbm.at[cid])

    return k(ewb, p, src, dst, zeros)


def _sc_ep(h0, tw, rev, src2, dst, zeros, write_ht, chunk=40):
    e, dm = h0.shape
    n = zeros.shape[0]
    dw = dm // 2
    b_per_w = e // NW
    nch = b_per_w // chunk
    npair = nch // 2
    tail = nch % 2 == 1

    buf = lambda: pltpu.VMEM((chunk, dm), jnp.float32)
    wbuf = lambda: pltpu.VMEM((chunk, dm), jnp.float32)
    ibuf = lambda: pltpu.VMEM((chunk,), jnp.int32)
    sem = pltpu.SemaphoreType.DMA

    outs = [jax.ShapeDtypeStruct((NC, n, dm), jnp.float32)]
    if write_ht:
        outs = [jax.ShapeDtypeStruct((e, dm), jnp.float32)] + outs

    @functools.partial(
        pl.kernel,
        mesh=_sc_mesh(),
        out_type=tuple(outs),
        scratch_types=(
            [buf() for _ in range(2)]
            + [wbuf() for _ in range(4)]
            + [ibuf() for _ in range(6)]
            + [pltpu.VMEM_SHARED((n, dm), jnp.float32)]
            + [sem] * 16
        ),
    )
    def k(h0_hbm, t_hbm, rev_hbm, src_hbm, dst_hbm, zeros_hbm, *rest):
        if write_ht:
            ht_hbm, parts_hbm = rest[0], rest[1]
            rest = rest[2:]
        else:
            parts_hbm = rest[0]
            ht_hbm = None
            rest = rest[1:]
        (h0_a, h0_b, tr_a, tr_b, ts_a, ts_b,
         ir_a, ir_b, is_a, is_b, id_a, id_b, acc,
         s_ha, s_hb, s_ra, s_rb, s_sa, s_sb,
         s_ira, s_irb, s_isa, s_isb, s_ida, s_idb,
         s_sta, s_stb, s_sca, s_scb) = rest

        cid = lax.axis_index("c")
        sid = lax.axis_index("s")
        wid = sid * NC + cid
        base = wid * b_per_w

        A = (h0_a, tr_a, ts_a, ir_a, is_a, id_a,
             s_ha, s_ra, s_sa, s_ira, s_isa, s_ida, s_sta, s_sca)
        B = (h0_b, tr_b, ts_b, ir_b, is_b, id_b,
             s_hb, s_rb, s_sb, s_irb, s_isb, s_idb, s_stb, s_scb)

        def off(c):
            return pl.multiple_of(base + c * chunk, 8)

        def ld_h0(c, S):
            return _mac(h0_hbm.at[pl.ds(off(c), chunk)], S[0], S[6])

        def ld_ir(c, S):
            return _mac(rev_hbm.at[pl.ds(off(c), chunk)], S[3], S[9])

        def ld_is(c, S):
            return _mac(src_hbm.at[pl.ds(off(c), chunk)], S[4], S[10])

        def ld_id(c, S):
            return _mac(dst_hbm.at[pl.ds(off(c), chunk)], S[5], S[11])

        def gat_r(S):
            return _mac(t_hbm.at[S[3]], S[1], S[7])

        def gat_s(S):
            return _mac(t_hbm.at[S[4]], S[2], S[8])

        def st(c, S):
            return _mac(S[0], ht_hbm.at[pl.ds(off(c), chunk)], S[12])

        def sca(S):
            return _mac(S[0], acc.at[S[5]], S[13])

        def compute(S):
            h0_v, tr_v, ts_v = S[0], S[1], S[2]

            def row(r, carry):
                for g in range(dm // 16):
                    sl = pl.ds(g * 16, 16)
                    v = h0_v[r, sl] + ts_v[r, sl] - tr_v[r, sl]
                    h0_v[r, sl] = jnp.maximum(v, 0.0)
                return carry

            lax.fori_loop(0, chunk, row, 0)

        def start_loads(c, S):
            ld_h0(c, S).start()
            ld_ir(c, S).start()
            ld_is(c, S).start()
            ld_id(c, S).start()

        @pl.when(sid == 0)
        def _():
            pltpu.sync_copy(zeros_hbm, acc)

        plsc.subcore_barrier()
        start_loads(0, A)

        def body(j, carry):
            c0 = 2 * j
            c1 = c0 + 1
            ld_ir(c0, A).wait()
            ld_is(c0, A).wait()
            gat_r(A).start()
            gat_s(A).start()

            @pl.when(j >= 1)
            def _():
                if write_ht:
                    st(c1 - 2, B).wait()
                sca(B).wait()

            start_loads(c1, B)
            ld_h0(c0, A).wait()
            gat_r(A).wait()
            gat_s(A).wait()
            compute(A)
            ld_id(c0, A).wait()
            if write_ht:
                st(c0, A).start()
            sca(A).start(add=True)
            ld_ir(c1, B).wait()
            ld_is(c1, B).wait()
            gat_r(B).start()
            gat_s(B).start()
            ld_h0(c1, B).wait()
            gat_r(B).wait()
            gat_s(B).wait()
            compute(B)

            @pl.when(c0 + 2 < nch)
            def _():
                if write_ht:
                    st(c0, A).wait()
                sca(A).wait()
                start_loads(c0 + 2, A)

            ld_id(c1, B).wait()
            if write_ht:
                st(c1, B).start()
            sca(B).start(add=True)
            return carry

        lax.fori_loop(0, npair, body, 0)
        if tail:
            c = nch - 1
            ld_ir(c, A).wait()
            ld_is(c, A).wait()
            gat_r(A).start()
            gat_s(A).start()
            ld_h0(c, A).wait()
            gat_r(A).wait()
            gat_s(A).wait()
            compute(A)
            ld_id(c, A).wait()
            if write_ht:
                st(c, A).start()
            sca(A).start(add=True)
            if write_ht:
                st(c, A).wait()
            sca(A).wait()
        else:
            if write_ht:
                st(nch - 2, A).wait()
            sca(A).wait()
        if write_ht:
            st(nch - 2 if tail else nch - 1, B).wait()
        sca(B).wait()
        plsc.subcore_barrier()

        @pl.when(sid == 0)
        def _():
            pltpu.sync_copy(acc, parts_hbm.at[cid])

    return k(h0, tw, rev, src2, dst, zeros)


def _tc_a(ea_t, we_t, b_i, bk=2560):
    de, e = ea_t.shape

    def body(ea_ref, w_ref, b_ref, o_ref):
        o_ref[...] = lax.dot_general(
            ea_ref[...], w_ref[...], (((0,), (0,)), ((), ())),
            preferred_element_type=jnp.float32) + b_ref[...]

    return pl.pallas_call(
        body,
        grid=(e // bk,),
        in_specs=[
            pl.BlockSpec((de, bk), lambda i: (0, i)),
            pl.BlockSpec((de, 128), lambda i: (0, 0)),
            pl.BlockSpec((1, 128), lambda i: (0, 0)),
        ],
        out_specs=pl.BlockSpec((bk, 128), lambda i: (i, 0)),
        out_shape=jax.ShapeDtypeStruct((e, 128), jnp.float32),
        compiler_params=pltpu.CompilerParams(dimension_semantics=("parallel",)),
    )(ea_t, we_t, b_i)


def _tc_p(x, wx_t, bn=1000):
    n, d = x.shape

    def body(x_ref, w_ref, o_ref):
        o_ref[...] = jnp.dot(x_ref[...], w_ref[...],
                             preferred_element_type=jnp.float32)

    return pl.pallas_call(
        body,
        grid=(n // bn,),
        in_specs=[pl.BlockSpec((bn, d), lambda i: (i, 0)),
                  pl.BlockSpec((d, 128), lambda i: (0, 0))],
        out_specs=pl.BlockSpec((bn, 128), lambda i: (i, 0)),
        out_shape=jax.ShapeDtypeStruct((n, 128), jnp.float32),
        compiler_params=pltpu.CompilerParams(dimension_semantics=("parallel",)),
    )(x, wx_t)


def _tc_t(h, p0, p1, wh_t, b_h, relu_in, bk=2000):
    e = h.shape[0]
    n = p0.shape[0]
    ge = e // bk
    gn = n // bk

    def body(h_ref, p0_ref, p1_ref, w_ref, b_ref, t_ref):
        i = pl.program_id(0)

        @pl.when(i < ge)
        def _():
            hv = h_ref[...]
            if relu_in:
                hv = jnp.maximum(hv, 0.0)
            t_ref[...] = jnp.dot(hv, w_ref[...],
                                 preferred_element_type=jnp.float32)

        @pl.when(i >= ge)
        def _():
            t_ref[...] = jnp.dot(p0_ref[...] + p1_ref[...], w_ref[...],
                                 preferred_element_type=jnp.float32) + b_ref[...]

    return pl.pallas_call(
        body,
        grid=(ge + gn,),
        in_specs=[
            pl.BlockSpec((bk, 128), lambda i: (jnp.minimum(i, ge - 1), 0)),
            pl.BlockSpec((bk, 128), lambda i: (jnp.maximum(i - ge, 0), 0)),
            pl.BlockSpec((bk, 128), lambda i: (jnp.maximum(i - ge, 0), 0)),
            pl.BlockSpec((128, 128), lambda i: (0, 0)),
            pl.BlockSpec((1, 128), lambda i: (0, 0)),
        ],
        out_specs=pl.BlockSpec((bk, 128), lambda i: (i, 0)),
        out_shape=jax.ShapeDtypeStruct((e + n, 128), jnp.float32),
        compiler_params=pltpu.CompilerParams(dimension_semantics=("arbitrary",)),
    )(h, p0, p1, wh_t, b_h)


def _tc_out(x, p0, p1, wox_t, wom_t, b_o, bn=1000):
    n = x.shape[0]

    def body(x_ref, p0_ref, p1_ref, wx_ref, wm_ref, b_ref, o_ref):
        agg = p0_ref[...] + p1_ref[...]
        xv = x_ref[...]
        m = jnp.where(jnp.sum(agg, axis=1, keepdims=True) == 0.0, xv, agg)
        acc = jnp.dot(xv, wx_ref[...], preferred_element_type=jnp.float32)
        acc += jnp.dot(m, wm_ref[...], preferred_element_type=jnp.float32)
        o_ref[...] = jnp.maximum(acc + b_ref[...], 0.0)

    return pl.pallas_call(
        body,
        grid=(n // bn,),
        in_specs=[pl.BlockSpec((bn, 128), lambda i: (i, 0)),
                  pl.BlockSpec((bn, 128), lambda i: (i, 0)),
                  pl.BlockSpec((bn, 128), lambda i: (i, 0)),
                  pl.BlockSpec((128, 128), lambda i: (0, 0)),
                  pl.BlockSpec((128, 128), lambda i: (0, 0)),
                  pl.BlockSpec((1, 128), lambda i: (0, 0))],
        out_specs=pl.BlockSpec((bn, 128), lambda i: (i, 0)),
        out_shape=jax.ShapeDtypeStruct((n, 128), jnp.float32),
        compiler_params=pltpu.CompilerParams(dimension_semantics=("parallel",)),
    )(x, p0, p1, wox_t, wom_t, b_o)


def kernel(x, edge_attr, W_i, b_i, W_h, b_h, W_o, b_o, edge_index, rev_edge_index):
    n, d = x.shape
    e = edge_attr.shape[0]
    hid = W_h.shape[0]
    src = edge_index[0]
    dst = edge_index[1]
    src2 = src + jnp.int32(e)

    wx_t = W_i[:, :d].T
    we_t = W_i[:, d:].T
    wh_t = W_h.T
    wox_t = W_o[:, :d].T
    wom_t = W_o[:, d:].T
    b_i2 = b_i.reshape(1, hid)
    b_h2 = b_h.reshape(1, hid)
    b_o2 = b_o.reshape(1, hid)
    zeros_n = jnp.zeros((n, hid), jnp.float32)

    ewb = _tc_a(edge_attr.T, we_t, b_i2)
    p = _tc_p(x, wx_t)
    h0, parts = _sc_ep_a(ewb, p, src, dst, zeros_n)

    t1 = _tc_t(h0, parts[0], parts[1], wh_t, b_h2, relu_in=True)
    ht, parts = _sc_ep(h0, t1, rev_edge_index, src2, dst, zeros_n,
                       write_ht=True)

    t2 = _tc_t(ht, parts[0], parts[1], wh_t, b_h2, relu_in=False)
    parts, = _sc_ep(h0, t2, rev_edge_index, src2, dst, zeros_n,
                    write_ht=False)

    return _tc_out(x, parts[0], parts[1], wox_t, wom_t, b_o2)
